# baseline (speedup 1.0000x reference)
import functools

import jax
import jax.numpy as jnp
from jax import lax
from jax.experimental import pallas as pl
from jax.experimental.pallas import tpu as pltpu
from jax.experimental.pallas import tpu_sc as plsc

B, S = 4, 2048
D = 128
EPS = 1e-07

NC, NS = 2, 16
NW = NC * NS
NT = B * S
TPW = NT // NW
CHUNK = 128
NCH = TPW // CHUNK

TPB = 256
NB = NT // TPB


def _gather_body(ids_hbm, wt_hbm, out_hbm, idx_v, rows_v, sem):
    c = lax.axis_index("c")
    s = lax.axis_index("s")
    wid = s * NC + c
    base = wid * TPW

    pltpu.sync_copy(ids_hbm.at[wid], idx_v)
    copies = [
        pltpu.async_copy(wt_hbm.at[idx_v.at[j]],
                         rows_v.at[pl.ds(j * CHUNK, CHUNK)], sem)
        for j in range(NCH)
    ]
    for cp in copies:
        cp.wait()
    pltpu.sync_copy(rows_v, out_hbm.at[pl.ds(base, TPW)])


def _sc_gather(ids, word_table):
    run = functools.partial(
        pl.kernel,
        out_type=jax.ShapeDtypeStruct((NT, D), jnp.float32),
        mesh=plsc.VectorSubcoreMesh(core_axis_name="c", subcore_axis_name="s"),
        scratch_types=[
            pltpu.VMEM((NCH, CHUNK), jnp.int32),
            pltpu.VMEM((TPW, D), jnp.float32),
            pltpu.SemaphoreType.DMA,
        ],
    )(_gather_body)
    return run(ids, word_table)


def _ln_body(rows_ref, pos_ref, tt_ref, g_ref, b_ref, o_ref):
    x = rows_ref[...] + pos_ref[...] + tt_ref[0:1, :]
    mean = jnp.mean(x, axis=-1, keepdims=True)
    xc = x - mean
    var = jnp.mean(xc * xc, axis=-1, keepdims=True)
    o_ref[...] = xc * lax.rsqrt(var + EPS) * g_ref[0:1, :] + b_ref[0:1, :]


def _tc_layernorm(rows, pos_table, tt_table, gamma, beta):
    return pl.pallas_call(
        _ln_body,
        grid=(NB,),
        in_specs=[
            pl.BlockSpec((TPB, D), lambda i: (i, 0)),
            pl.BlockSpec((TPB, D), lambda i: (lax.rem(i, S // TPB), 0)),
            pl.BlockSpec((2, D), lambda i: (0, 0)),
            pl.BlockSpec((1, D), lambda i: (0, 0)),
            pl.BlockSpec((1, D), lambda i: (0, 0)),
        ],
        out_specs=pl.BlockSpec((TPB, D), lambda i: (i, 0)),
        out_shape=jax.ShapeDtypeStruct((NT, D), jnp.float32),
    )(rows, pos_table, tt_table, gamma, beta)


@jax.jit
def kernel(input_ids, word_table, pos_table, tt_table, gamma, beta):
    ids = input_ids.astype(jnp.int32).reshape(NW, NCH, CHUNK)
    rows = _sc_gather(ids, word_table)
    out = _tc_layernorm(rows, pos_table, tt_table,
                        gamma.reshape(1, D), beta.reshape(1, D))
    return out.reshape(B, S, D)


# speedup vs baseline: 1.3601x; 1.3601x over previous
import functools

import jax
import jax.numpy as jnp
from jax import lax
from jax.experimental import pallas as pl
from jax.experimental.pallas import tpu as pltpu
from jax.experimental.pallas import tpu_sc as plsc

B, S = 4, 2048
D = 128
EPS = 1e-07

NC, NS = 2, 16
NW = NC * NS
NT = B * S
TPW = NT // NW
CHUNK = 128
NCH = TPW // CHUNK

TPB = 1024
NB = NT // TPB
SPB = S // TPB


def _gather_body(ids_hbm, wt_hbm, out_hbm, idx_v, rows_v, sem):
    c = lax.axis_index("c")
    s = lax.axis_index("s")
    wid = s * NC + c
    base = wid * TPW
    b = wid // (S // TPW)
    pbase = lax.rem(wid, S // TPW) * TPW

    for j in range(NCH):
        pltpu.sync_copy(ids_hbm.at[b, pl.ds(pbase + j * CHUNK, CHUNK)],
                        idx_v.at[j])
    copies = [
        pltpu.async_copy(wt_hbm.at[idx_v.at[j]],
                         rows_v.at[pl.ds(j * CHUNK, CHUNK)], sem)
        for j in range(NCH)
    ]
    for cp in copies:
        cp.wait()
    pltpu.sync_copy(rows_v, out_hbm.at[pl.ds(base, TPW)])


def _sc_gather(ids, word_table):
    run = functools.partial(
        pl.kernel,
        out_type=jax.ShapeDtypeStruct((NT, D), jnp.float32),
        mesh=plsc.VectorSubcoreMesh(core_axis_name="c", subcore_axis_name="s"),
        scratch_types=[
            pltpu.VMEM((NCH, CHUNK), jnp.int32),
            pltpu.VMEM((TPW, D), jnp.float32),
            pltpu.SemaphoreType.DMA,
        ],
    )(_gather_body)
    return run(ids, word_table)


def _ln_body(rows_ref, pos_ref, tt_ref, g_ref, b_ref, o_ref):
    x = rows_ref[...] + pos_ref[...] + tt_ref[0:1, :]
    mean = jnp.mean(x, axis=-1, keepdims=True)
    xc = x - mean
    var = jnp.mean(xc * xc, axis=-1, keepdims=True)
    o_ref[0, ...] = xc * lax.rsqrt(var + EPS) * g_ref[0:1, :] + b_ref[0:1, :]


def _tc_layernorm(rows, pos_table, tt_table, gamma, beta):
    return pl.pallas_call(
        _ln_body,
        grid=(NB,),
        in_specs=[
            pl.BlockSpec((TPB, D), lambda i: (i, 0)),
            pl.BlockSpec((TPB, D), lambda i: (lax.rem(i, SPB), 0)),
            pl.BlockSpec((2, D), lambda i: (0, 0)),
            pl.BlockSpec((1, D), lambda i: (0, 0)),
            pl.BlockSpec((1, D), lambda i: (0, 0)),
        ],
        out_specs=pl.BlockSpec((1, TPB, D), lambda i: (i // SPB, lax.rem(i, SPB), 0)),
        out_shape=jax.ShapeDtypeStruct((B, S, D), jnp.float32),
    )(rows, pos_table, tt_table, gamma, beta)


@jax.jit
def kernel(input_ids, word_table, pos_table, tt_table, gamma, beta):
    ids = input_ids.astype(jnp.int32)
    rows = _sc_gather(ids, word_table)
    return _tc_layernorm(rows, pos_table, tt_table,
                         gamma.reshape(1, D), beta.reshape(1, D))


# speedup vs baseline: 1.4889x; 1.0947x over previous
import functools

import jax
import jax.numpy as jnp
from jax import lax
from jax.experimental import pallas as pl
from jax.experimental.pallas import tpu as pltpu
from jax.experimental.pallas import tpu_sc as plsc

B, S = 4, 2048
D = 128
EPS = 1e-07

NC, NS = 2, 16
NW = NC * NS
NT = B * S
TPW = NT // NW
CHUNK = 128
NCH = TPW // CHUNK

TPB = 2048
NB = NT // TPB
SPB = S // TPB


def _gather_body(ids_hbm, wt_hbm, out_hbm, idx_v, rows_v, sem):
    c = lax.axis_index("c")
    s = lax.axis_index("s")
    wid = s * NC + c
    base = wid * TPW
    b = wid // (S // TPW)
    pbase = lax.rem(wid, S // TPW) * TPW

    for j in range(NCH):
        pltpu.sync_copy(ids_hbm.at[b, pl.ds(pbase + j * CHUNK, CHUNK)],
                        idx_v.at[j])
    copies = [
        pltpu.async_copy(wt_hbm.at[idx_v.at[j]],
                         rows_v.at[pl.ds(j * CHUNK, CHUNK)], sem)
        for j in range(NCH)
    ]
    for cp in copies:
        cp.wait()
    pltpu.sync_copy(rows_v, out_hbm.at[pl.ds(base, TPW)])


def _sc_gather(ids, word_table):
    run = functools.partial(
        pl.kernel,
        out_type=jax.ShapeDtypeStruct((NT, D), jnp.float32),
        mesh=plsc.VectorSubcoreMesh(core_axis_name="c", subcore_axis_name="s"),
        scratch_types=[
            pltpu.VMEM((NCH, CHUNK), jnp.int32),
            pltpu.VMEM((TPW, D), jnp.float32),
            pltpu.SemaphoreType.DMA,
        ],
    )(_gather_body)
    return run(ids, word_table)


def _ln_body(rows_ref, pos_ref, tt_ref, g_ref, b_ref, o_ref):
    x = rows_ref[...] + pos_ref[...] + tt_ref[0:1, :]
    mean = jnp.mean(x, axis=-1, keepdims=True)
    xc = x - mean
    var = jnp.mean(xc * xc, axis=-1, keepdims=True)
    o_ref[0, ...] = xc * lax.rsqrt(var + EPS) * g_ref[0:1, :] + b_ref[0:1, :]


def _tc_layernorm(rows, pos_table, tt_table, gamma, beta):
    return pl.pallas_call(
        _ln_body,
        grid=(NB,),
        in_specs=[
            pl.BlockSpec((TPB, D), lambda i: (i, 0)),
            pl.BlockSpec((TPB, D), lambda i: (lax.rem(i, SPB), 0)),
            pl.BlockSpec((2, D), lambda i: (0, 0)),
            pl.BlockSpec((1, D), lambda i: (0, 0)),
            pl.BlockSpec((1, D), lambda i: (0, 0)),
        ],
        out_specs=pl.BlockSpec((1, TPB, D), lambda i: (i // SPB, lax.rem(i, SPB), 0)),
        out_shape=jax.ShapeDtypeStruct((B, S, D), jnp.float32),
    )(rows, pos_table, tt_table, gamma, beta)


@jax.jit
def kernel(input_ids, word_table, pos_table, tt_table, gamma, beta):
    ids = input_ids.astype(jnp.int32)
    rows = _sc_gather(ids, word_table)
    return _tc_layernorm(rows, pos_table, tt_table,
                         gamma.reshape(1, D), beta.reshape(1, D))
